# baseline (speedup 1.0000x reference)
import functools

import jax
import jax.numpy as jnp
from jax import lax
from jax.experimental import pallas as pl
from jax.experimental.pallas import tpu as pltpu
from jax.experimental.pallas import tpu_sc as plsc

CLS = 100000
FEAT = 64
BATCH = 16384
NC = 2
NS = 16
L = 16
NW = NC * NS
BW = BATCH // NW
HPT = BATCH // NS
CPAD = 100096
ZCHUNK = CPAD // NS

_MAGIC = 0x5F3759DF


def _sc_body(xs_hbm, ys2_hbm, center_hbm, part_hbm,
             idx_v, idxh_v, ones_v, zeros_v, cnt_v, xs_v, rows_v, acc_v,
             hist_sh, sem, sem2, sem3):
    cid = lax.axis_index("c")
    sid = lax.axis_index("s")
    wid = sid * NC + cid

    pltpu.sync_copy(ys2_hbm.at[pl.ds(wid * (BW // 128), BW // 128)], idx_v)
    pltpu.sync_copy(ys2_hbm.at[pl.ds(sid * (HPT // 128), HPT // 128)], idxh_v)
    @pl.loop(0, ZCHUNK, step=L)
    def _(i):
        zeros_v[pl.ds(i, L)] = jnp.zeros((L,), jnp.float32)

    cp_zero = pltpu.async_copy(zeros_v,
                               hist_sh.at[pl.ds(sid * ZCHUNK, ZCHUNK)], sem3)

    cp_xs = pltpu.async_copy(xs_hbm.at[pl.ds(wid * BW * FEAT, BW * FEAT)],
                             xs_v, sem)

    @plsc.parallel_loop(0, BW // L)
    def _(g):
        y16 = idx_v[g >> 3, pl.ds((g & 7) * L, L)]
        for k in range(L):
            r = g * L + k
            y = y16[k]
            pltpu.async_copy(center_hbm.at[y >> 3, pl.ds(y & 7, 1)],
                             rows_v.at[r >> 3, pl.ds(r & 7, 1)], sem2)

    @pl.loop(0, 128, step=L)
    def _(i):
        ones_v[pl.ds(i, L)] = jnp.ones((L,), jnp.float32)

    acc_v[...] = jnp.zeros((L,), jnp.float32)

    cp_zero.wait()
    plsc.subcore_barrier()

    for j in range(HPT // 128):
        pltpu.sync_copy(ones_v, hist_sh.at[idxh_v.at[j]], add=True)

    plsc.subcore_barrier()

    for j in range(BW // 128):
        pltpu.sync_copy(hist_sh.at[idx_v.at[j]], cnt_v.at[j])

    pltpu.make_async_copy(center_hbm.at[pl.ds(0, BW // 8)], rows_v,
                          sem2).wait()
    cp_xs.wait()

    @plsc.parallel_loop(0, BW // L, carry=jnp.zeros((L,), jnp.float32))
    def acc_fin(g, acc):
        c16 = cnt_v[g >> 3, pl.ds((g & 7) * L, L)]
        w16 = 0.5 / (c16 + 1.0)
        for k in range(L):
            r = g * L + k
            o = r * FEAT
            x0 = xs_v[pl.ds(o, L)]
            x1 = xs_v[pl.ds(o + L, L)]
            x2 = xs_v[pl.ds(o + 2 * L, L)]
            x3 = xs_v[pl.ds(o + 3 * L, L)]
            sq = (x0 * x0 + x1 * x1) + (x2 * x2 + x3 * x3)
            s = jnp.sum(sq)
            sb = jnp.broadcast_to(s, (L,))
            iv = plsc.bitcast(sb, jnp.int32)
            y = plsc.bitcast(jnp.int32(_MAGIC) - (iv >> 1), jnp.float32)
            hs = 0.5 * sb
            y = y * (1.5 - hs * y * y)
            y = y * (1.5 - hs * y * y)
            rinv = jnp.where(sb < 1e-24, jnp.float32(1e12), y)
            wb = jnp.broadcast_to(w16[k], (L,))
            rh, rl = r >> 3, r & 7
            d0 = x0 * rinv - rows_v[rh, rl, pl.ds(0, L)]
            d1 = x1 * rinv - rows_v[rh, rl, pl.ds(L, L)]
            d2 = x2 * rinv - rows_v[rh, rl, pl.ds(2 * L, L)]
            d3 = x3 * rinv - rows_v[rh, rl, pl.ds(3 * L, L)]
            dsq = (d0 * d0 + d1 * d1) + (d2 * d2 + d3 * d3)
            acc = acc + wb * dsq
        return acc

    acc_v[...] = acc_fin
    pltpu.sync_copy(acc_v, part_hbm.at[wid])


_cp = pltpu.CompilerParams(
    needs_layout_passes=False,
    use_tc_tiling_on_sc=True,
)

_sc_loss = functools.partial(
    pl.kernel,
    out_type=jax.ShapeDtypeStruct((NW, L), jnp.float32),
    mesh=plsc.VectorSubcoreMesh(
        core_axis_name="c", subcore_axis_name="s",
        num_cores=NC, num_subcores=NS),
    compiler_params=_cp,
    scratch_types=[
        pltpu.VMEM((BW // 128, 128), jnp.int32),
        pltpu.VMEM((HPT // 128, 128), jnp.int32),
        pltpu.VMEM((128,), jnp.float32),
        pltpu.VMEM((ZCHUNK,), jnp.float32),
        pltpu.VMEM((BW // 128, 128), jnp.float32),
        pltpu.VMEM((BW * FEAT,), jnp.float32),
        pltpu.VMEM((BW // 8, 8, FEAT), jnp.float32),
        pltpu.VMEM((L,), jnp.float32),
        pltpu.VMEM_SHARED((CPAD,), jnp.float32),
        pltpu.SemaphoreType.DMA,
        pltpu.SemaphoreType.DMA,
        pltpu.SemaphoreType.DMA,
    ],
)(_sc_body)


def _tc_sum_body(p_ref, o_ref):
    o_ref[...] = jnp.sum(p_ref[...])[None, None]


def _tc_sum(parts):
    return pl.pallas_call(
        _tc_sum_body,
        out_shape=jax.ShapeDtypeStruct((1, 1), jnp.float32),
    )(parts)


def kernel(xs, ys, center):
    ys2 = ys.astype(jnp.int32).reshape(128, 128)
    xs_flat = xs.reshape(BATCH * FEAT)
    center3 = center.reshape(CLS // 8, 8, FEAT)
    parts = _sc_loss(xs_flat, ys2, center3)
    return _tc_sum(parts)[0, 0]


# speedup vs baseline: 1.1033x; 1.1033x over previous
import functools

import jax
import jax.numpy as jnp
from jax import lax
from jax.experimental import pallas as pl
from jax.experimental.pallas import tpu as pltpu
from jax.experimental.pallas import tpu_sc as plsc

CLS = 100000
FEAT = 64
BATCH = 16384
NC = 2
NS = 16
L = 16
NW = NC * NS
BW = BATCH // NW
HPT = BATCH // NS
CPAD = 100096
ZCHUNK = CPAD // NS

_MAGIC = 0x5F3759DF


def _sc_body(xs_hbm, ys2_hbm, center_hbm, part_hbm,
             idx_v, idxh_v, ones_v, zeros_v, cnt_v, xs_v, rows_v, acc_v,
             hist_sh, sem, sem2, sem3):
    cid = lax.axis_index("c")
    sid = lax.axis_index("s")
    wid = sid * NC + cid

    pltpu.sync_copy(ys2_hbm.at[pl.ds(wid * (BW // 128), BW // 128)], idx_v)
    pltpu.sync_copy(ys2_hbm.at[pl.ds(sid * (HPT // 128), HPT // 128)], idxh_v)
    @pl.loop(0, ZCHUNK, step=L)
    def _(i):
        zeros_v[pl.ds(i, L)] = jnp.zeros((L,), jnp.float32)

    cp_zero = pltpu.async_copy(zeros_v,
                               hist_sh.at[pl.ds(sid * ZCHUNK, ZCHUNK)], sem3)

    cp_xs = pltpu.async_copy(xs_hbm.at[pl.ds(wid * BW * FEAT, BW * FEAT)],
                             xs_v, sem)

    @pl.loop(0, BW // L)
    def _(g):
        y16 = idx_v[g >> 3, pl.ds((g & 7) * L, L)]
        for k in range(L):
            r = g * L + k
            y = y16[k]
            pltpu.async_copy(center_hbm.at[y >> 3, pl.ds(y & 7, 1)],
                             rows_v.at[r >> 3, pl.ds(r & 7, 1)], sem2)

    @pl.loop(0, 128, step=L)
    def _(i):
        ones_v[pl.ds(i, L)] = jnp.ones((L,), jnp.float32)

    acc_v[...] = jnp.zeros((L,), jnp.float32)

    cp_zero.wait()
    plsc.subcore_barrier()

    for j in range(HPT // 128):
        pltpu.sync_copy(ones_v, hist_sh.at[idxh_v.at[j]], add=True)

    plsc.subcore_barrier()

    for j in range(BW // 128):
        pltpu.sync_copy(hist_sh.at[idx_v.at[j]], cnt_v.at[j])

    pltpu.make_async_copy(center_hbm.at[pl.ds(0, BW // 8)], rows_v,
                          sem2).wait()
    cp_xs.wait()

    @pl.loop(0, BW // L)
    def _(g):
        c16 = cnt_v[g >> 3, pl.ds((g & 7) * L, L)]
        w16 = 0.5 / (c16 + 1.0)
        acc = acc_v[...]
        for k in range(L):
            r = g * L + k
            o = r * FEAT
            x0 = xs_v[pl.ds(o, L)]
            x1 = xs_v[pl.ds(o + L, L)]
            x2 = xs_v[pl.ds(o + 2 * L, L)]
            x3 = xs_v[pl.ds(o + 3 * L, L)]
            sq = (x0 * x0 + x1 * x1) + (x2 * x2 + x3 * x3)
            s = jnp.sum(sq)
            sb = jnp.broadcast_to(s, (L,))
            iv = plsc.bitcast(sb, jnp.int32)
            y = plsc.bitcast(jnp.int32(_MAGIC) - (iv >> 1), jnp.float32)
            hs = 0.5 * sb
            y = y * (1.5 - hs * y * y)
            y = y * (1.5 - hs * y * y)
            rinv = jnp.where(sb < 1e-24, jnp.float32(1e12), y)
            wb = jnp.broadcast_to(w16[k], (L,))
            rh, rl = r >> 3, r & 7
            d0 = x0 * rinv - rows_v[rh, rl, pl.ds(0, L)]
            d1 = x1 * rinv - rows_v[rh, rl, pl.ds(L, L)]
            d2 = x2 * rinv - rows_v[rh, rl, pl.ds(2 * L, L)]
            d3 = x3 * rinv - rows_v[rh, rl, pl.ds(3 * L, L)]
            dsq = (d0 * d0 + d1 * d1) + (d2 * d2 + d3 * d3)
            acc = acc + wb * dsq
        acc_v[...] = acc

    pltpu.sync_copy(acc_v, part_hbm.at[wid])


_cp = pltpu.CompilerParams(
    needs_layout_passes=False,
    use_tc_tiling_on_sc=True,
)

_sc_loss = functools.partial(
    pl.kernel,
    out_type=jax.ShapeDtypeStruct((NW, L), jnp.float32),
    mesh=plsc.VectorSubcoreMesh(
        core_axis_name="c", subcore_axis_name="s",
        num_cores=NC, num_subcores=NS),
    compiler_params=_cp,
    scratch_types=[
        pltpu.VMEM((BW // 128, 128), jnp.int32),
        pltpu.VMEM((HPT // 128, 128), jnp.int32),
        pltpu.VMEM((128,), jnp.float32),
        pltpu.VMEM((ZCHUNK,), jnp.float32),
        pltpu.VMEM((BW // 128, 128), jnp.float32),
        pltpu.VMEM((BW * FEAT,), jnp.float32),
        pltpu.VMEM((BW // 8, 8, FEAT), jnp.float32),
        pltpu.VMEM((L,), jnp.float32),
        pltpu.VMEM_SHARED((CPAD,), jnp.float32),
        pltpu.SemaphoreType.DMA,
        pltpu.SemaphoreType.DMA,
        pltpu.SemaphoreType.DMA,
    ],
)(_sc_body)


def _tc_sum_body(p_ref, o_ref):
    o_ref[...] = jnp.sum(p_ref[...])[None, None]


def _tc_sum(parts):
    return pl.pallas_call(
        _tc_sum_body,
        out_shape=jax.ShapeDtypeStruct((1, 1), jnp.float32),
    )(parts)


def kernel(xs, ys, center):
    ys2 = ys.astype(jnp.int32).reshape(128, 128)
    xs_flat = xs.reshape(BATCH * FEAT)
    center3 = center.reshape(CLS // 8, 8, FEAT)
    parts = _sc_loss(xs_flat, ys2, center3)
    return _tc_sum(parts)[0, 0]
